# baseline (speedup 1.0000x reference)
import functools

import jax
import jax.numpy as jnp
from jax import lax
from jax.experimental import pallas as pl
from jax.experimental.pallas import tpu as pltpu
from jax.experimental.pallas import tpu_sc as plsc

E = 8
TOPK = 2
D = 1024
D_FFN = 2048
TOTAL = E * D_FFN
T = 2048
N = T * TOPK
BM = 256
NB = N // BM + E
NP = NB * BM
CH = 512
NW = 32
TPW = T // NW


def _router_body(x_ref, wr_ref, pos0_ref, pos1_ref, w1_ref, w2_ref,
                 be_ref, cnt_ref, z_ref, lb_ref):
    x = x_ref[...]
    wr = wr_ref[...]
    logits = jax.lax.dot_general(x, wr, (((1,), (1,)), ((), ())),
                                 preferred_element_type=jnp.float32)
    m = jnp.max(logits, axis=-1, keepdims=True)
    ex = jnp.exp(logits - m)
    se = jnp.sum(ex, axis=-1, keepdims=True)
    probs = ex / se
    lse = m[:, 0] + jnp.log(se[:, 0])
    z_ref[...] = jnp.mean(lse * lse).reshape(1, 1)

    iota = jax.lax.broadcasted_iota(jnp.int32, (T, E), 1)
    p1 = jnp.max(probs, axis=-1, keepdims=True)
    i1 = jnp.min(jnp.where(probs == p1, iota, E), axis=-1)
    masked = jnp.where(iota == i1[:, None], -jnp.inf, probs)
    p2 = jnp.max(masked, axis=-1, keepdims=True)
    i2 = jnp.min(jnp.where(masked == p2, iota, E), axis=-1)
    s = p1[:, 0] + p2[:, 0]
    w1_ref[...] = jnp.broadcast_to((p1[:, 0] / s)[:, None], (T, 16))
    w2_ref[...] = jnp.broadcast_to((p2[:, 0] / s)[:, None], (T, 16))

    o0 = (iota == i1[:, None]).astype(jnp.float32)
    o1 = (iota == i2[:, None]).astype(jnp.float32)
    oh = o0 + o1
    cnt = jnp.sum(oh, axis=0, keepdims=True)
    cnt_ref[...] = cnt.astype(jnp.int32)
    p_i = jnp.mean(probs, axis=0)
    lb_ref[...] = (E * jnp.sum((cnt[0] / N) * p_i)).reshape(1, 1)

    ri = jax.lax.broadcasted_iota(jnp.int32, (CH, CH), 0)
    ci = jax.lax.broadcasted_iota(jnp.int32, (CH, CH), 1)
    ltri = (ri > ci).astype(jnp.float32)
    pchunks = []
    carry = jnp.zeros((1, E), jnp.float32)
    for c in range(T // CH):
        blk = oh[c * CH:(c + 1) * CH]
        pchunks.append(jnp.dot(ltri, blk, preferred_element_type=jnp.float32)
                       + carry)
        carry = carry + jnp.sum(blk, axis=0, keepdims=True)
    pfx = jnp.concatenate(pchunks, axis=0)

    pc = jnp.ceil(cnt * (1.0 / BM)) * BM
    tri8r = jax.lax.broadcasted_iota(jnp.int32, (E, E), 0)
    tri8c = jax.lax.broadcasted_iota(jnp.int32, (E, E), 1)
    tri8 = (tri8r <= tri8c).astype(jnp.float32)
    cum_pc = jnp.dot(pc, tri8, preferred_element_type=jnp.float32)
    poff = cum_pc - pc

    pos0 = jnp.sum((poff + pfx) * o0, axis=1)
    pos1 = jnp.sum((poff + pfx) * o1, axis=1)
    pos0_ref[...] = pos0.astype(jnp.int32)[:, None]
    pos1_ref[...] = pos1.astype(jnp.int32)[:, None]

    bid = jax.lax.broadcasted_iota(jnp.int32, (NB, E), 0).astype(jnp.float32) * BM
    be = jnp.minimum(jnp.sum((bid >= cum_pc).astype(jnp.int32), axis=1), E - 1)
    e_iota = jax.lax.broadcasted_iota(jnp.int32, (NB, E), 1)
    nz = (pc > 0).astype(jnp.int32)
    seg = jnp.sum((e_iota < be[:, None]) * nz, axis=1)
    par = seg & 1
    oh_be = (e_iota == be[:, None]).astype(jnp.float32)
    nz_be = jnp.sum(nz.astype(jnp.float32) * oh_be, axis=1)
    fb = ((bid[:, 0] == jnp.sum(poff * oh_be, axis=1)) & (nz_be > 0)
          ).astype(jnp.int32)
    nxte = jnp.min(jnp.where((e_iota > be[:, None]) & (nz > 0), e_iota, E),
                   axis=1)
    pf = fb * (nxte < E).astype(jnp.int32)
    nxte = jnp.minimum(nxte, E - 1)
    sc = jnp.concatenate([be[None, :], par[None, :], fb[None, :],
                          pf[None, :], nxte[None, :]], axis=0)
    be_ref[...] = sc


def _dispatch_scatter(xf, p0, p1):
    mesh = plsc.VectorSubcoreMesh(core_axis_name="c", subcore_axis_name="s")

    @functools.partial(
        pl.kernel, mesh=mesh,
        out_type=jax.ShapeDtypeStruct((NP, D), jnp.float32),
        scratch_types=[
            pltpu.VMEM((TPW, D), jnp.float32),
            pltpu.VMEM((TPW,), jnp.int32),
            pltpu.VMEM((TPW,), jnp.int32),
            pltpu.SemaphoreType.DMA,
        ],
    )
    def body(xf_hbm, p0_hbm, p1_hbm, xs_hbm, rows_v, i0_v, i1_v, sem):
        wid = lax.axis_index("s") * 2 + lax.axis_index("c")
        base = wid * TPW
        pltpu.sync_copy(xf_hbm.at[pl.ds(base, TPW)], rows_v)
        pltpu.sync_copy(p0_hbm.at[pl.ds(base, TPW)], i0_v)
        pltpu.sync_copy(p1_hbm.at[pl.ds(base, TPW)], i1_v)
        c0 = pltpu.async_copy(rows_v, xs_hbm.at[i0_v], sem)
        c1 = pltpu.async_copy(rows_v, xs_hbm.at[i1_v], sem)
        c0.wait()
        c1.wait()

    return body(xf, p0, p1)


CC = 16
NCH = TPW // CC


def _combine(ys, p0r, p1r, w0r, w1r):
    mesh = plsc.VectorSubcoreMesh(core_axis_name="c", subcore_axis_name="s")

    @functools.partial(
        pl.kernel, mesh=mesh,
        out_type=jax.ShapeDtypeStruct((T, D), jnp.float32),
        scratch_types=[
            pltpu.VMEM((NCH, CC), jnp.int32),
            pltpu.VMEM((NCH, CC), jnp.int32),
            pltpu.VMEM((TPW, 16), jnp.float32),
            pltpu.VMEM((TPW, 16), jnp.float32),
            pltpu.VMEM((CC, D), jnp.float32),
            pltpu.VMEM((CC, D), jnp.float32),
            pltpu.VMEM((CC, D), jnp.float32),
            pltpu.VMEM((CC, D), jnp.float32),
            pltpu.VMEM((CC, D), jnp.float32),
            pltpu.VMEM((CC, D), jnp.float32),
            pltpu.SemaphoreType.DMA,
            pltpu.SemaphoreType.DMA,
            pltpu.SemaphoreType.DMA,
            pltpu.SemaphoreType.DMA,
            pltpu.SemaphoreType.DMA,
            pltpu.SemaphoreType.DMA,
        ],
    )
    def body(ys_hbm, p0_hbm, p1_hbm, w0_hbm, w1_hbm, out_hbm,
             i0_v, i1_v, w0_v, w1_v, r0a, r0b, r1a, r1b, oa, ob,
             sg0a, sg0b, sg1a, sg1b, soa, sob):
        wid = lax.axis_index("s") * 2 + lax.axis_index("c")
        base = wid * TPW
        pltpu.sync_copy(p0_hbm.at[wid], i0_v)
        pltpu.sync_copy(p1_hbm.at[wid], i1_v)
        pltpu.sync_copy(w0_hbm.at[wid], w0_v)
        pltpu.sync_copy(w1_hbm.at[wid], w1_v)

        r0 = (r0a, r0b)
        r1 = (r1a, r1b)
        ob_ = (oa, ob)
        sg0 = (sg0a, sg0b)
        sg1 = (sg1a, sg1b)
        so = (soa, sob)

        def gather(c):
            u = c % 2
            g0 = pltpu.async_copy(ys_hbm.at[i0_v.at[c]], r0[u], sg0[u])
            g1 = pltpu.async_copy(ys_hbm.at[i1_v.at[c]], r1[u], sg1[u])
            return g0, g1

        pend = {0: gather(0), 1: None}
        outw = {0: None, 1: None}
        for c in range(NCH):
            u = c % 2
            if c + 1 < NCH:
                pend[(c + 1) % 2] = gather(c + 1)
            g0, g1 = pend[u]
            g0.wait()
            g1.wait()
            if outw[u] is not None:
                outw[u].wait()

            ws0 = [w0_v[c * CC + r, :] for r in range(CC)]
            ws1 = [w1_v[c * CC + r, :] for r in range(CC)]

            def jbody(j, _):
                for r in range(CC):
                    sl = pl.ds(j * 16, 16)
                    ob_[u][r, sl] = (r0[u][r, sl] * ws0[r]
                                     + r1[u][r, sl] * ws1[r])
                return 0
            lax.fori_loop(0, D // 16, jbody, 0, unroll=12)

            ow = pltpu.make_async_copy(
                ob_[u], out_hbm.at[pl.ds(base + c * CC, CC)], so[u])
            ow.start()
            outw[u] = ow
        outw[0].wait()
        outw[1].wait()

    return body(ys, p0r, p1r, w0r, w1r)


def _ffn_body(sc_ref, xs_ref, w1_hbm, w2_hbm, ys_ref,
              w1b0, w1b1, w2b0, w2b1, s10, s11, s20, s21):
    b = pl.program_id(0)
    e = sc_ref[0, b]
    par = sc_ref[1, b]
    fb = sc_ref[2, b]
    pf = sc_ref[3, b]
    nxte = sc_ref[4, b]

    def w_copies(ee, w1b, w2b, sa, sb):
        c1 = pltpu.make_async_copy(
            w1_hbm.at[:, pl.ds(ee * D_FFN, D_FFN)], w1b, sa)
        c2 = pltpu.make_async_copy(
            w2_hbm.at[pl.ds(ee * D_FFN, D_FFN), :], w2b, sb)
        return c1, c2

    @pl.when(b == 0)
    def _():
        c1, c2 = w_copies(e, w1b0, w2b0, s10, s20)
        c1.start()
        c2.start()

    @pl.when((fb == 1) & (par == 0))
    def _():
        c1, c2 = w_copies(e, w1b0, w2b0, s10, s20)
        c1.wait()
        c2.wait()

    @pl.when((fb == 1) & (par == 1))
    def _():
        c1, c2 = w_copies(e, w1b1, w2b1, s11, s21)
        c1.wait()
        c2.wait()

    @pl.when((pf == 1) & (par == 0))
    def _():
        c1, c2 = w_copies(nxte, w1b1, w2b1, s11, s21)
        c1.start()
        c2.start()

    @pl.when((pf == 1) & (par == 1))
    def _():
        c1, c2 = w_copies(nxte, w1b0, w2b0, s10, s20)
        c1.start()
        c2.start()

    def compute(w1b, w2b):
        h = jnp.dot(xs_ref[...], w1b[...], preferred_element_type=jnp.float32)
        h = jax.nn.gelu(h)
        ys_ref[...] = jnp.dot(h, w2b[...], preferred_element_type=jnp.float32)

    @pl.when(par == 0)
    def _():
        compute(w1b0, w2b0)

    @pl.when(par != 0)
    def _():
        compute(w1b1, w2b1)


def _grouped_ffn(sc, xs, W1, W2):
    grid_spec = pltpu.PrefetchScalarGridSpec(
        num_scalar_prefetch=1,
        grid=(NB,),
        in_specs=[
            pl.BlockSpec((BM, D), lambda b, sc_s: (b, 0)),
            pl.BlockSpec(memory_space=pl.ANY),
            pl.BlockSpec(memory_space=pl.ANY),
        ],
        out_specs=pl.BlockSpec((BM, D), lambda b, sc_s: (b, 0)),
        scratch_shapes=[
            pltpu.VMEM((D, D_FFN), jnp.float32),
            pltpu.VMEM((D, D_FFN), jnp.float32),
            pltpu.VMEM((D_FFN, D), jnp.float32),
            pltpu.VMEM((D_FFN, D), jnp.float32),
            pltpu.SemaphoreType.DMA,
            pltpu.SemaphoreType.DMA,
            pltpu.SemaphoreType.DMA,
            pltpu.SemaphoreType.DMA,
        ],
    )
    return pl.pallas_call(
        _ffn_body,
        grid_spec=grid_spec,
        out_shape=jax.ShapeDtypeStruct((NP, D), jnp.float32),
        compiler_params=pltpu.CompilerParams(
            dimension_semantics=("arbitrary",)),
    )(sc, xs, W1, W2)


def kernel(x, Wr, W1, W2):
    xf = x.reshape(-1, D)

    pos0, pos1, w1, w2, sc, cnt, z, lb = pl.pallas_call(
        _router_body,
        out_shape=[
            jax.ShapeDtypeStruct((T, 1), jnp.int32),
            jax.ShapeDtypeStruct((T, 1), jnp.int32),
            jax.ShapeDtypeStruct((T, 16), jnp.float32),
            jax.ShapeDtypeStruct((T, 16), jnp.float32),
            jax.ShapeDtypeStruct((5, NB), jnp.int32),
            jax.ShapeDtypeStruct((1, E), jnp.int32),
            jax.ShapeDtypeStruct((1, 1), jnp.float32),
            jax.ShapeDtypeStruct((1, 1), jnp.float32),
        ],
    )(xf, Wr)

    p0 = pos0.reshape(-1)
    p1 = pos1.reshape(-1)
    xs = _dispatch_scatter(xf, p0, p1)
    ys = _grouped_ffn(sc, xs, W1, W2)

    out2 = _combine(ys,
                    p0.reshape(NW, NCH, CC), p1.reshape(NW, NCH, CC),
                    w1.reshape(NW, TPW, 16), w2.reshape(NW, TPW, 16))
    out = out2.reshape(1, T, D)
    f_i = cnt[0].astype(jnp.float32) / N
    return (out, z[0, 0], lb[0, 0], f_i)


# speedup vs baseline: 1.0634x; 1.0634x over previous
import functools

import jax
import jax.numpy as jnp
from jax import lax
from jax.experimental import pallas as pl
from jax.experimental.pallas import tpu as pltpu
from jax.experimental.pallas import tpu_sc as plsc

E = 8
TOPK = 2
D = 1024
D_FFN = 2048
TOTAL = E * D_FFN
T = 2048
N = T * TOPK
BM = 256
NB = N // BM + E
NP = NB * BM
CH = 512
NW = 32
TPW = T // NW


def _router_body(x_ref, wr_ref, pos0_ref, pos1_ref, w1_ref, w2_ref,
                 be_ref, cnt_ref, z_ref, lb_ref):
    x = x_ref[...]
    wr = wr_ref[...]
    logits = jax.lax.dot_general(x, wr, (((1,), (1,)), ((), ())),
                                 preferred_element_type=jnp.float32)
    m = jnp.max(logits, axis=-1, keepdims=True)
    ex = jnp.exp(logits - m)
    se = jnp.sum(ex, axis=-1, keepdims=True)
    probs = ex / se
    lse = m[:, 0] + jnp.log(se[:, 0])
    z_ref[...] = jnp.mean(lse * lse).reshape(1, 1)

    iota = jax.lax.broadcasted_iota(jnp.int32, (T, E), 1)
    p1 = jnp.max(probs, axis=-1, keepdims=True)
    i1 = jnp.min(jnp.where(probs == p1, iota, E), axis=-1)
    masked = jnp.where(iota == i1[:, None], -jnp.inf, probs)
    p2 = jnp.max(masked, axis=-1, keepdims=True)
    i2 = jnp.min(jnp.where(masked == p2, iota, E), axis=-1)
    s = p1[:, 0] + p2[:, 0]
    w1_ref[...] = jnp.broadcast_to((p1[:, 0] / s)[:, None], (T, 16))
    w2_ref[...] = jnp.broadcast_to((p2[:, 0] / s)[:, None], (T, 16))

    o0 = (iota == i1[:, None]).astype(jnp.float32)
    o1 = (iota == i2[:, None]).astype(jnp.float32)
    oh = o0 + o1
    cnt = jnp.sum(oh, axis=0, keepdims=True)
    cnt_ref[...] = cnt.astype(jnp.int32)
    p_i = jnp.mean(probs, axis=0)
    lb_ref[...] = (E * jnp.sum((cnt[0] / N) * p_i)).reshape(1, 1)

    csum = oh
    k = 1
    while k < T:
        shifted = jnp.concatenate(
            [jnp.zeros((k, E), jnp.float32), csum[:T - k]], axis=0)
        csum = csum + shifted
        k *= 2
    pfx = csum - oh

    pc = jnp.ceil(cnt * (1.0 / BM)) * BM
    tri8r = jax.lax.broadcasted_iota(jnp.int32, (E, E), 0)
    tri8c = jax.lax.broadcasted_iota(jnp.int32, (E, E), 1)
    tri8 = (tri8r <= tri8c).astype(jnp.float32)
    cum_pc = jnp.dot(pc, tri8, preferred_element_type=jnp.float32)
    poff = cum_pc - pc

    pos0 = jnp.sum((poff + pfx) * o0, axis=1)
    pos1 = jnp.sum((poff + pfx) * o1, axis=1)
    pos0_ref[...] = pos0.astype(jnp.int32)[:, None]
    pos1_ref[...] = pos1.astype(jnp.int32)[:, None]

    bid = jax.lax.broadcasted_iota(jnp.int32, (NB, E), 0).astype(jnp.float32) * BM
    be = jnp.minimum(jnp.sum((bid >= cum_pc).astype(jnp.int32), axis=1), E - 1)
    e_iota = jax.lax.broadcasted_iota(jnp.int32, (NB, E), 1)
    nz = (pc > 0).astype(jnp.int32)
    seg = jnp.sum((e_iota < be[:, None]) * nz, axis=1)
    par = seg & 1
    oh_be = (e_iota == be[:, None]).astype(jnp.float32)
    nz_be = jnp.sum(nz.astype(jnp.float32) * oh_be, axis=1)
    fb = ((bid[:, 0] == jnp.sum(poff * oh_be, axis=1)) & (nz_be > 0)
          ).astype(jnp.int32)
    nxte = jnp.min(jnp.where((e_iota > be[:, None]) & (nz > 0), e_iota, E),
                   axis=1)
    pf = fb * (nxte < E).astype(jnp.int32)
    nxte = jnp.minimum(nxte, E - 1)
    sc = jnp.concatenate([be[None, :], par[None, :], fb[None, :],
                          pf[None, :], nxte[None, :]], axis=0)
    be_ref[...] = sc


def _dispatch_scatter(xf, p0, p1):
    mesh = plsc.VectorSubcoreMesh(core_axis_name="c", subcore_axis_name="s")

    @functools.partial(
        pl.kernel, mesh=mesh,
        out_type=jax.ShapeDtypeStruct((NP, D), jnp.float32),
        scratch_types=[
            pltpu.VMEM((TPW, D), jnp.float32),
            pltpu.VMEM((TPW,), jnp.int32),
            pltpu.VMEM((TPW,), jnp.int32),
            pltpu.SemaphoreType.DMA,
        ],
    )
    def body(xf_hbm, p0_hbm, p1_hbm, xs_hbm, rows_v, i0_v, i1_v, sem):
        wid = lax.axis_index("s") * 2 + lax.axis_index("c")
        base = wid * TPW
        pltpu.sync_copy(xf_hbm.at[pl.ds(base, TPW)], rows_v)
        pltpu.sync_copy(p0_hbm.at[pl.ds(base, TPW)], i0_v)
        pltpu.sync_copy(p1_hbm.at[pl.ds(base, TPW)], i1_v)
        c0 = pltpu.async_copy(rows_v, xs_hbm.at[i0_v], sem)
        c1 = pltpu.async_copy(rows_v, xs_hbm.at[i1_v], sem)
        c0.wait()
        c1.wait()

    return body(xf, p0, p1)


CC = 16
NCH = TPW // CC


def _combine(ys, p0r, p1r, w0r, w1r):
    mesh = plsc.VectorSubcoreMesh(core_axis_name="c", subcore_axis_name="s")

    @functools.partial(
        pl.kernel, mesh=mesh,
        out_type=jax.ShapeDtypeStruct((T, D), jnp.float32),
        scratch_types=[
            pltpu.VMEM((NCH, CC), jnp.int32),
            pltpu.VMEM((NCH, CC), jnp.int32),
            pltpu.VMEM((TPW, 16), jnp.float32),
            pltpu.VMEM((TPW, 16), jnp.float32),
            pltpu.VMEM((CC, D), jnp.float32),
            pltpu.VMEM((CC, D), jnp.float32),
            pltpu.VMEM((CC, D), jnp.float32),
            pltpu.VMEM((CC, D), jnp.float32),
            pltpu.VMEM((CC, D), jnp.float32),
            pltpu.VMEM((CC, D), jnp.float32),
            pltpu.SemaphoreType.DMA,
            pltpu.SemaphoreType.DMA,
            pltpu.SemaphoreType.DMA,
            pltpu.SemaphoreType.DMA,
            pltpu.SemaphoreType.DMA,
            pltpu.SemaphoreType.DMA,
        ],
    )
    def body(ys_hbm, p0_hbm, p1_hbm, w0_hbm, w1_hbm, out_hbm,
             i0_v, i1_v, w0_v, w1_v, r0a, r0b, r1a, r1b, oa, ob,
             sg0a, sg0b, sg1a, sg1b, soa, sob):
        wid = lax.axis_index("s") * 2 + lax.axis_index("c")
        base = wid * TPW
        pltpu.sync_copy(p0_hbm.at[wid], i0_v)
        pltpu.sync_copy(p1_hbm.at[wid], i1_v)
        pltpu.sync_copy(w0_hbm.at[wid], w0_v)
        pltpu.sync_copy(w1_hbm.at[wid], w1_v)

        r0 = (r0a, r0b)
        r1 = (r1a, r1b)
        ob_ = (oa, ob)
        sg0 = (sg0a, sg0b)
        sg1 = (sg1a, sg1b)
        so = (soa, sob)

        def gather(c):
            u = c % 2
            g0 = pltpu.async_copy(ys_hbm.at[i0_v.at[c]], r0[u], sg0[u])
            g1 = pltpu.async_copy(ys_hbm.at[i1_v.at[c]], r1[u], sg1[u])
            return g0, g1

        pend = {0: gather(0), 1: None}
        outw = {0: None, 1: None}
        for c in range(NCH):
            u = c % 2
            if c + 1 < NCH:
                pend[(c + 1) % 2] = gather(c + 1)
            g0, g1 = pend[u]
            g0.wait()
            g1.wait()
            if outw[u] is not None:
                outw[u].wait()

            ws0 = [w0_v[c * CC + r, :] for r in range(CC)]
            ws1 = [w1_v[c * CC + r, :] for r in range(CC)]

            def jbody(j, _):
                for r in range(CC):
                    sl = pl.ds(j * 16, 16)
                    ob_[u][r, sl] = (r0[u][r, sl] * ws0[r]
                                     + r1[u][r, sl] * ws1[r])
                return 0
            lax.fori_loop(0, D // 16, jbody, 0, unroll=8)

            ow = pltpu.make_async_copy(
                ob_[u], out_hbm.at[pl.ds(base + c * CC, CC)], so[u])
            ow.start()
            outw[u] = ow
        outw[0].wait()
        outw[1].wait()

    return body(ys, p0r, p1r, w0r, w1r)


def _ffn_body(sc_ref, xs_ref, w1_hbm, w2_hbm, ys_ref,
              w1b0, w1b1, w2b0, w2b1, s10, s11, s20, s21):
    b = pl.program_id(0)
    e = sc_ref[0, b]
    par = sc_ref[1, b]
    fb = sc_ref[2, b]
    pf = sc_ref[3, b]
    nxte = sc_ref[4, b]

    def w_copies(ee, w1b, w2b, sa, sb):
        c1 = pltpu.make_async_copy(
            w1_hbm.at[:, pl.ds(ee * D_FFN, D_FFN)], w1b, sa)
        c2 = pltpu.make_async_copy(
            w2_hbm.at[pl.ds(ee * D_FFN, D_FFN), :], w2b, sb)
        return c1, c2

    @pl.when(b == 0)
    def _():
        c1, c2 = w_copies(e, w1b0, w2b0, s10, s20)
        c1.start()
        c2.start()

    @pl.when((fb == 1) & (par == 0))
    def _():
        c1, c2 = w_copies(e, w1b0, w2b0, s10, s20)
        c1.wait()
        c2.wait()

    @pl.when((fb == 1) & (par == 1))
    def _():
        c1, c2 = w_copies(e, w1b1, w2b1, s11, s21)
        c1.wait()
        c2.wait()

    @pl.when((pf == 1) & (par == 0))
    def _():
        c1, c2 = w_copies(nxte, w1b1, w2b1, s11, s21)
        c1.start()
        c2.start()

    @pl.when((pf == 1) & (par == 1))
    def _():
        c1, c2 = w_copies(nxte, w1b0, w2b0, s10, s20)
        c1.start()
        c2.start()

    def compute(w1b, w2b):
        h = jnp.dot(xs_ref[...], w1b[...], preferred_element_type=jnp.float32)
        h = jax.nn.gelu(h)
        ys_ref[...] = jnp.dot(h, w2b[...], preferred_element_type=jnp.float32)

    @pl.when(par == 0)
    def _():
        compute(w1b0, w2b0)

    @pl.when(par != 0)
    def _():
        compute(w1b1, w2b1)


def _grouped_ffn(sc, xs, W1, W2):
    grid_spec = pltpu.PrefetchScalarGridSpec(
        num_scalar_prefetch=1,
        grid=(NB,),
        in_specs=[
            pl.BlockSpec((BM, D), lambda b, sc_s: (b, 0)),
            pl.BlockSpec(memory_space=pl.ANY),
            pl.BlockSpec(memory_space=pl.ANY),
        ],
        out_specs=pl.BlockSpec((BM, D), lambda b, sc_s: (b, 0)),
        scratch_shapes=[
            pltpu.VMEM((D, D_FFN), jnp.float32),
            pltpu.VMEM((D, D_FFN), jnp.float32),
            pltpu.VMEM((D_FFN, D), jnp.float32),
            pltpu.VMEM((D_FFN, D), jnp.float32),
            pltpu.SemaphoreType.DMA,
            pltpu.SemaphoreType.DMA,
            pltpu.SemaphoreType.DMA,
            pltpu.SemaphoreType.DMA,
        ],
    )
    return pl.pallas_call(
        _ffn_body,
        grid_spec=grid_spec,
        out_shape=jax.ShapeDtypeStruct((NP, D), jnp.float32),
        compiler_params=pltpu.CompilerParams(
            dimension_semantics=("arbitrary",)),
    )(sc, xs, W1, W2)


def kernel(x, Wr, W1, W2):
    xf = x.reshape(-1, D)

    pos0, pos1, w1, w2, sc, cnt, z, lb = pl.pallas_call(
        _router_body,
        out_shape=[
            jax.ShapeDtypeStruct((T, 1), jnp.int32),
            jax.ShapeDtypeStruct((T, 1), jnp.int32),
            jax.ShapeDtypeStruct((T, 16), jnp.float32),
            jax.ShapeDtypeStruct((T, 16), jnp.float32),
            jax.ShapeDtypeStruct((5, NB), jnp.int32),
            jax.ShapeDtypeStruct((1, E), jnp.int32),
            jax.ShapeDtypeStruct((1, 1), jnp.float32),
            jax.ShapeDtypeStruct((1, 1), jnp.float32),
        ],
    )(xf, Wr)

    p0 = pos0.reshape(-1)
    p1 = pos1.reshape(-1)
    xs = _dispatch_scatter(xf, p0, p1)
    ys = _grouped_ffn(sc, xs, W1, W2)

    out2 = _combine(ys,
                    p0.reshape(NW, NCH, CC), p1.reshape(NW, NCH, CC),
                    w1.reshape(NW, TPW, 16), w2.reshape(NW, TPW, 16))
    out = out2.reshape(1, T, D)
    f_i = cnt[0].astype(jnp.float32) / N
    return (out, z[0, 0], lb[0, 0], f_i)
